# baseline (speedup 1.0000x reference)
import functools

import jax
import jax.numpy as jnp
from jax import lax
from jax.experimental import pallas as pl
from jax.experimental.pallas import tpu as pltpu
from jax.experimental.pallas import tpu_sc as plsc

VOCAB = 100000
F_DIM = 64
HIST = 50
MU = 3.5
NORM = float(HIST) ** (-0.5)
NCHUNK = F_DIM // 16
NTILE = 16
KMAX = (HIST + NTILE - 1) // NTILE


def _base_lane(idx):
    return pl.multiple_of((idx >> 7) << 7, 128), idx & 127


NROW = NTILE + 3


def _svdpp_body(user_hbm, item_hbm, imp_hbm, ub_hbm, ib_hbm, P_hbm, Q_hbm,
                Y_hbm, out_hbm, xch_hbm, user_v, item_v, imp_v, bu_v, bi_v,
                pu_v, qi_v, rows_v, part_v, bvec_v, fin_v, res_v,
                sem0, sem1, sem2, sem3):
    cid = lax.axis_index("c")
    sid = lax.axis_index("s")

    @pl.when(cid == 0)
    def _():
        iota = lax.iota(jnp.int32, 16)

        def col_chunks(ref3, slot, lane):
            slots = jnp.full((16,), slot, jnp.int32)
            lanes = jnp.full((16,), lane, jnp.int32)
            return [
                plsc.load_gather(ref3, [slots, iota + (c * 16), lanes])
                for c in range(NCHUNK)
            ]

        c0 = pltpu.async_copy(user_hbm, user_v.at[pl.ds(0, 1)], sem0)
        c1 = pltpu.async_copy(item_hbm, item_v.at[pl.ds(0, 1)], sem1)
        c2 = pltpu.async_copy(imp_hbm, imp_v.at[pl.ds(0, HIST)], sem2)
        c2.wait()

        valids, lanes_k = [], []
        ycopies = []
        for k in range(KMAX):
            j = sid + (k * NTILE)
            valid = j < HIST
            ivec = plsc.load_gather(
                imp_v, [jnp.full((16,), jnp.where(valid, j, 0), jnp.int32)])
            idx = jnp.where(valid, ivec[0], 0)
            base, lane = _base_lane(idx)
            valids.append(valid)
            lanes_k.append(lane)
            ycopies.append(pltpu.async_copy(
                Y_hbm.at[:, pl.ds(base, 128)], rows_v.at[k], sem3))

        c0.wait()
        c1.wait()

        @pl.when(sid == 2)
        def _():
            u = user_v[...][0]
            base, lane = _base_lane(u)
            gP = pltpu.async_copy(P_hbm.at[:, pl.ds(base, 128)],
                                  pu_v.at[0], sem0)
            gb = pltpu.async_copy(ub_hbm.at[:, pl.ds(base, 128)], bu_v, sem1)
            gP.wait()
            gb.wait()
            pc = col_chunks(pu_v, 0, lane)
            for c in range(NCHUNK):
                part_v[pl.ds(F_DIM + c * 16, 16)] = pc[c]
            bvec_v[...] = plsc.load_gather(
                bu_v, [jnp.zeros((16,), jnp.int32),
                       jnp.full((16,), lane, jnp.int32)])
            pltpu.sync_copy(part_v.at[pl.ds(F_DIM, F_DIM)],
                            xch_hbm.at[pl.ds(NTILE * F_DIM, F_DIM)])
            pltpu.sync_copy(bvec_v,
                            xch_hbm.at[pl.ds((NTILE + 2) * F_DIM, 16)])

        @pl.when(sid == 3)
        def _():
            it = item_v[...][0]
            base, lane = _base_lane(it)
            gQ = pltpu.async_copy(Q_hbm.at[:, pl.ds(base, 128)],
                                  qi_v.at[0], sem0)
            gb = pltpu.async_copy(ib_hbm.at[:, pl.ds(base, 128)], bi_v, sem1)
            gQ.wait()
            gb.wait()
            qc = col_chunks(qi_v, 0, lane)
            for c in range(NCHUNK):
                part_v[pl.ds(F_DIM + c * 16, 16)] = qc[c]
            bvec_v[...] = plsc.load_gather(
                bi_v, [jnp.zeros((16,), jnp.int32),
                       jnp.full((16,), lane, jnp.int32)])
            pltpu.sync_copy(part_v.at[pl.ds(F_DIM, F_DIM)],
                            xch_hbm.at[pl.ds((NTILE + 1) * F_DIM, F_DIM)])
            pltpu.sync_copy(bvec_v,
                            xch_hbm.at[pl.ds((NTILE + 2) * F_DIM + 16, 16)])

        zero = jnp.zeros((16,), jnp.float32)
        acc = [zero] * NCHUNK
        for k in range(KMAX):
            ycopies[k].wait()
            row = col_chunks(rows_v, k, lanes_k[k])
            for c in range(NCHUNK):
                acc[c] = acc[c] + jnp.where(valids[k], row[c], zero)
        for c in range(NCHUNK):
            part_v[pl.ds(c * 16, 16)] = acc[c]
        pltpu.sync_copy(part_v.at[pl.ds(0, F_DIM)],
                        xch_hbm.at[pl.ds(pl.multiple_of(sid * F_DIM, 8), F_DIM)])

        plsc.subcore_barrier()

        @pl.when(sid == 0)
        def _():
            pltpu.sync_copy(xch_hbm, fin_v)
            total = None
            for c in range(NCHUNK):
                sl = slice(c * 16, (c + 1) * 16)
                a = fin_v[pl.ds(c * 16, 16)]
                for w in range(1, NTILE):
                    a = a + fin_v[pl.ds(w * F_DIM + c * 16, 16)]
                t = fin_v[pl.ds(NTILE * F_DIM + c * 16, 16)] * (
                    fin_v[pl.ds((NTILE + 1) * F_DIM + c * 16, 16)] + NORM * a)
                total = t if total is None else total + t
            s = total[0]
            for i in range(1, 16):
                s = s + total[i]
            bu = fin_v[pl.ds((NTILE + 2) * F_DIM, 16)][0]
            bi = fin_v[pl.ds((NTILE + 2) * F_DIM + 16, 16)][0]
            r = MU + bu + bi + s
            res_v[...] = jnp.full((16,), r, jnp.float32)
            pltpu.sync_copy(res_v.at[pl.ds(0, 1)], out_hbm)


_svdpp = functools.partial(
    pl.kernel,
    out_type=(jax.ShapeDtypeStruct((1,), jnp.float32),
              jax.ShapeDtypeStruct((NROW * F_DIM,), jnp.float32)),
    mesh=plsc.VectorSubcoreMesh(core_axis_name="c", subcore_axis_name="s"),
    compiler_params=pltpu.CompilerParams(
        needs_layout_passes=False, disable_bounds_checks=True),
    scratch_types=[
        pltpu.VMEM((16,), jnp.int32),
        pltpu.VMEM((16,), jnp.int32),
        pltpu.VMEM((64,), jnp.int32),
        pltpu.VMEM((1, 128), jnp.float32),
        pltpu.VMEM((1, 128), jnp.float32),
        pltpu.VMEM((1, F_DIM, 128), jnp.float32),
        pltpu.VMEM((1, F_DIM, 128), jnp.float32),
        pltpu.VMEM((KMAX, F_DIM, 128), jnp.float32),
        pltpu.VMEM((2 * F_DIM,), jnp.float32),
        pltpu.VMEM((16,), jnp.float32),
        pltpu.VMEM((NROW * F_DIM,), jnp.float32),
        pltpu.VMEM((16,), jnp.float32),
        pltpu.SemaphoreType.DMA,
        pltpu.SemaphoreType.DMA,
        pltpu.SemaphoreType.DMA,
        pltpu.SemaphoreType.DMA,
    ],
)(_svdpp_body)


def kernel(user, item, implicit_items, user_biases, item_biases, P, Q, Y):
    out, _ = _svdpp(
        user.astype(jnp.int32),
        item.astype(jnp.int32),
        implicit_items.astype(jnp.int32),
        user_biases.T,
        item_biases.T,
        P.T,
        Q.T,
        Y.T,
    )
    return out


# speedup vs baseline: 1.0018x; 1.0018x over previous
import functools

import jax
import jax.numpy as jnp
from jax import lax
from jax.experimental import pallas as pl
from jax.experimental.pallas import tpu as pltpu
from jax.experimental.pallas import tpu_sc as plsc

VOCAB = 100000
F_DIM = 64
HIST = 50
MU = 3.5
NORM = float(HIST) ** (-0.5)
NCHUNK = F_DIM // 16
NTILE = 16
KMAX = (HIST + NTILE - 1) // NTILE


def _base_lane(idx):
    return pl.multiple_of((idx >> 7) << 7, 128), idx & 127


NROW = NTILE + 3


def _svdpp_body(user_hbm, item_hbm, imp_hbm, ub_hbm, ib_hbm, P_hbm, Q_hbm,
                Y_hbm, out_hbm, xch_hbm, user_v, item_v, imp_v, bu_v, bi_v,
                pu_v, qi_v, rows_v, part_v, bvec_v, fin_v, res_v,
                sem0, sem1, sem2, sem3):
    cid = lax.axis_index("c")
    sid = lax.axis_index("s")

    @pl.when(cid == 0)
    def _():
        iota = lax.iota(jnp.int32, 16)

        def col_chunks(ref3, slot, lane):
            slots = jnp.full((16,), slot, jnp.int32)
            lanes = jnp.full((16,), lane, jnp.int32)
            return [
                plsc.load_gather(ref3, [slots, iota + (c * 16), lanes])
                for c in range(NCHUNK)
            ]

        c0 = pltpu.async_copy(user_hbm, user_v.at[pl.ds(0, 1)], sem0)
        c1 = pltpu.async_copy(item_hbm, item_v.at[pl.ds(0, 1)], sem1)
        c2 = pltpu.async_copy(imp_hbm, imp_v.at[pl.ds(0, HIST)], sem2)
        c2.wait()

        valids, lanes_k = [], []
        ycopies = []
        for k in range(KMAX):
            j = sid + (k * NTILE)
            valid = j < HIST
            ivec = plsc.load_gather(
                imp_v, [jnp.full((16,), jnp.where(valid, j, 0), jnp.int32)])
            idx = jnp.where(valid, ivec[0], 0)
            base, lane = _base_lane(idx)
            valids.append(valid)
            lanes_k.append(lane)
            ycopies.append(pltpu.async_copy(
                Y_hbm.at[:, pl.ds(base, 128)], rows_v.at[k], sem3))

        c0.wait()
        c1.wait()

        @pl.when(sid == 2)
        def _():
            u = user_v[...][0]
            base, lane = _base_lane(u)
            gP = pltpu.async_copy(P_hbm.at[:, pl.ds(base, 128)],
                                  pu_v.at[0], sem0)
            gb = pltpu.async_copy(ub_hbm.at[:, pl.ds(base, 128)], bu_v, sem1)
            gP.wait()
            gb.wait()
            pc = col_chunks(pu_v, 0, lane)
            for c in range(NCHUNK):
                part_v[pl.ds(F_DIM + c * 16, 16)] = pc[c]
            bvec_v[...] = plsc.load_gather(
                bu_v, [jnp.zeros((16,), jnp.int32),
                       jnp.full((16,), lane, jnp.int32)])
            pltpu.sync_copy(part_v.at[pl.ds(F_DIM, F_DIM)],
                            xch_hbm.at[pl.ds(NTILE * F_DIM, F_DIM)])
            pltpu.sync_copy(bvec_v,
                            xch_hbm.at[pl.ds((NTILE + 2) * F_DIM, 16)])

        @pl.when(sid == 3)
        def _():
            it = item_v[...][0]
            base, lane = _base_lane(it)
            gQ = pltpu.async_copy(Q_hbm.at[:, pl.ds(base, 128)],
                                  qi_v.at[0], sem0)
            gb = pltpu.async_copy(ib_hbm.at[:, pl.ds(base, 128)], bi_v, sem1)
            gQ.wait()
            gb.wait()
            qc = col_chunks(qi_v, 0, lane)
            for c in range(NCHUNK):
                part_v[pl.ds(F_DIM + c * 16, 16)] = qc[c]
            bvec_v[...] = plsc.load_gather(
                bi_v, [jnp.zeros((16,), jnp.int32),
                       jnp.full((16,), lane, jnp.int32)])
            pltpu.sync_copy(part_v.at[pl.ds(F_DIM, F_DIM)],
                            xch_hbm.at[pl.ds((NTILE + 1) * F_DIM, F_DIM)])
            pltpu.sync_copy(bvec_v,
                            xch_hbm.at[pl.ds((NTILE + 2) * F_DIM + 16, 16)])

        zero = jnp.zeros((16,), jnp.float32)
        acc = [zero] * NCHUNK
        for k in range(KMAX):
            ycopies[k].wait()
            row = col_chunks(rows_v, k, lanes_k[k])
            for c in range(NCHUNK):
                acc[c] = acc[c] + jnp.where(valids[k], row[c], zero)
        for c in range(NCHUNK):
            part_v[pl.ds(c * 16, 16)] = acc[c]
        pltpu.sync_copy(part_v.at[pl.ds(0, F_DIM)],
                        xch_hbm.at[pl.ds(pl.multiple_of(sid * F_DIM, 8), F_DIM)])

        plsc.subcore_barrier()

        @pl.when(sid == 0)
        def _():
            pltpu.sync_copy(xch_hbm, fin_v)
            total = None
            for c in range(NCHUNK):
                a = fin_v[pl.ds(c * 16, 16)]
                for w in range(1, NTILE):
                    a = a + fin_v[pl.ds(w * F_DIM + c * 16, 16)]
                t = fin_v[pl.ds(NTILE * F_DIM + c * 16, 16)] * (
                    fin_v[pl.ds((NTILE + 1) * F_DIM + c * 16, 16)] + NORM * a)
                total = t if total is None else total + t
            s = total[0]
            for i in range(1, 16):
                s = s + total[i]
            bu = fin_v[pl.ds((NTILE + 2) * F_DIM, 16)][0]
            bi = fin_v[pl.ds((NTILE + 2) * F_DIM + 16, 16)][0]
            r = MU + bu + bi + s
            res_v[...] = jnp.full((16,), r, jnp.float32)
            pltpu.sync_copy(res_v.at[pl.ds(0, 1)], out_hbm)


_svdpp = functools.partial(
    pl.kernel,
    out_type=(jax.ShapeDtypeStruct((1,), jnp.float32),
              jax.ShapeDtypeStruct((NROW * F_DIM,), jnp.float32)),
    mesh=plsc.VectorSubcoreMesh(core_axis_name="c", subcore_axis_name="s"),
    compiler_params=pltpu.CompilerParams(
        needs_layout_passes=False, disable_bounds_checks=True),
    scratch_types=[
        pltpu.VMEM((16,), jnp.int32),
        pltpu.VMEM((16,), jnp.int32),
        pltpu.VMEM((64,), jnp.int32),
        pltpu.VMEM((1, 128), jnp.float32),
        pltpu.VMEM((1, 128), jnp.float32),
        pltpu.VMEM((1, F_DIM, 128), jnp.float32),
        pltpu.VMEM((1, F_DIM, 128), jnp.float32),
        pltpu.VMEM((KMAX, F_DIM, 128), jnp.float32),
        pltpu.VMEM((2 * F_DIM,), jnp.float32),
        pltpu.VMEM((16,), jnp.float32),
        pltpu.VMEM((NROW * F_DIM,), jnp.float32),
        pltpu.VMEM((16,), jnp.float32),
        pltpu.SemaphoreType.DMA,
        pltpu.SemaphoreType.DMA,
        pltpu.SemaphoreType.DMA,
        pltpu.SemaphoreType.DMA,
    ],
)(_svdpp_body)


def kernel(user, item, implicit_items, user_biases, item_biases, P, Q, Y):
    out, _ = _svdpp(
        user.astype(jnp.int32),
        item.astype(jnp.int32),
        implicit_items.astype(jnp.int32),
        user_biases.T,
        item_biases.T,
        P.T,
        Q.T,
        Y.T,
    )
    return out
